# baseline (speedup 1.0000x reference)
import functools

import jax
import jax.numpy as jnp
from jax import lax
from jax.experimental import pallas as pl
from jax.experimental.pallas import tpu as pltpu
from jax.experimental.pallas import tpu_sc as plsc

_NNODES = 10000
_NEDGES = 320000
_NOPEN = 48
_H = 0.1

_EB = 2560
_F32 = jnp.float32


def _tvn(x, axis):
    x = x - jnp.mean(x, axis=axis, keepdims=True)
    return x / jnp.sqrt(jnp.sum(x * x, axis=axis, keepdims=True) + 1e-3)


def _node_body(xn_ref, k1_ref, k2_ref, out_ref):
    x = jnp.tanh(xn_ref[...])
    x = jnp.dot(k1_ref[...], x, preferred_element_type=_F32, precision=lax.Precision.HIGHEST)
    x = _tvn(x, 0)
    x = jnp.tanh(x)
    x = jnp.dot(k2_ref[...], x, preferred_element_type=_F32, precision=lax.Precision.HIGHEST)
    x = jnp.tanh(x)
    out_ref[...] = x.T


def _node_embed(xn, k1, k2):
    return pl.pallas_call(
        _node_body,
        out_shape=jax.ShapeDtypeStruct((_NNODES, 16), _F32),
    )(xn, k1, k2)


def _edge_body(xe_ref, k1_ref, k2_ref,
               w1a_ref, b1a_ref, w2a_ref, b2a_ref,
               w1b_ref, b1b_ref, w2b_ref, b2b_ref,
               xeout_ref, xep_ref, wg1_ref, wg2_ref):
    x = jnp.tanh(xe_ref[...])
    x = jnp.dot(k1_ref[...], x, preferred_element_type=_F32, precision=lax.Precision.HIGHEST)
    x = _tvn(x, 0)
    x = jnp.tanh(x)
    x = jnp.dot(k2_ref[...], x, preferred_element_type=_F32, precision=lax.Precision.HIGHEST)
    x = jnp.tanh(x)
    xeout_ref[...] = x
    xt = x.T
    xep_ref[...] = xt

    def mlp(w1, b1, w2, b2):
        h = jnp.tanh(jnp.dot(xt, w1[...].T, preferred_element_type=_F32, precision=lax.Precision.HIGHEST)
                     + b1[...])
        return jnp.dot(h, w2[...].T, preferred_element_type=_F32, precision=lax.Precision.HIGHEST) + b2[...]

    wg1_ref[...] = mlp(w1a_ref, b1a_ref, w2a_ref, b2a_ref) * xt
    wg2_ref[...] = mlp(w1b_ref, b1b_ref, w2b_ref, b2b_ref) * xt


def _edge_embed(xe, k1, k2, fa_w1, fa_b1, fa_w2, fa_b2):
    nb = _NEDGES // _EB
    wspec = [pl.BlockSpec(s, lambda i: (0, 0))
             for s in [(16, 16), (16, 16),
                       (25, 16), (1, 25), (16, 25), (1, 16),
                       (25, 16), (1, 25), (16, 25), (1, 16)]]
    return pl.pallas_call(
        _edge_body,
        grid=(nb,),
        in_specs=[pl.BlockSpec((16, _EB), lambda i: (0, i))] + wspec,
        out_specs=[pl.BlockSpec((16, _EB), lambda i: (0, i)),
                   pl.BlockSpec((_EB, 16), lambda i: (i, 0)),
                   pl.BlockSpec((_EB, 16), lambda i: (i, 0)),
                   pl.BlockSpec((_EB, 16), lambda i: (i, 0))],
        out_shape=[jax.ShapeDtypeStruct((16, _NEDGES), _F32),
                   jax.ShapeDtypeStruct((_NEDGES, 16), _F32),
                   jax.ShapeDtypeStruct((_NEDGES, 16), _F32),
                   jax.ShapeDtypeStruct((_NEDGES, 16), _F32)],
    )(xe, k1, k2,
      fa_w1[0], fa_b1[0][None], fa_w2[0], fa_b2[0][None],
      fa_w1[1], fa_b1[1][None], fa_w2[1], fa_b2[1][None])


def _edgea_body(xi_ref, xj_ref, xep_ref,
                g_w1, g_b1, g_w2, g_b2,
                a_w1, a_b1, a_w2, a_b2,
                d_w1, d_b1, d_w2, d_b2,
                v_w1, v_b1, v_w2, v_b2,
                ke1_ref, ke2_ref,
                wgd_ref, wga_ref):
    xt = xep_ref[...]

    def mlp(w1, b1, w2, b2):
        h = jnp.tanh(jnp.dot(xt, w1[...].T, preferred_element_type=_F32, precision=lax.Precision.HIGHEST)
                     + b1[...])
        return jnp.dot(h, w2[...].T, preferred_element_type=_F32, precision=lax.Precision.HIGHEST) + b2[...]

    gi = xi_ref[...]
    gj = xj_ref[...]
    grad_x = mlp(g_w1, g_b1, g_w2, g_b2) * (gi - gj)
    int_x = mlp(a_w1, a_b1, a_w2, a_b2) * (gi + gj) * 0.5
    e = jnp.concatenate([grad_x, int_x], axis=1)
    e = jnp.tanh(e)
    e = jnp.dot(e, ke1_ref[...].T, preferred_element_type=_F32, precision=lax.Precision.HIGHEST)
    e = _tvn(e, 1)
    e = jnp.tanh(e)
    e = jnp.dot(e, ke2_ref[...].T, preferred_element_type=_F32, precision=lax.Precision.HIGHEST)
    e = jnp.tanh(e)
    wgd_ref[...] = mlp(d_w1, d_b1, d_w2, d_b2) * e[:, :_NOPEN]
    wga_ref[...] = mlp(v_w1, v_b1, v_w2, v_b2) * e[:, _NOPEN:]


def _edge_layer(xi, xj, xep, fb, ke1, ke2):
    nb = _NEDGES // _EB
    wspec = []
    wargs = []
    for (w1, b1, w2, b2) in fb:
        wspec += [pl.BlockSpec((25, 16), lambda i: (0, 0)),
                  pl.BlockSpec((1, 25), lambda i: (0, 0)),
                  pl.BlockSpec((_NOPEN, 25), lambda i: (0, 0)),
                  pl.BlockSpec((1, _NOPEN), lambda i: (0, 0))]
        wargs += [w1, b1[None], w2, b2[None]]
    return pl.pallas_call(
        _edgea_body,
        grid=(nb,),
        in_specs=[pl.BlockSpec((_EB, _NOPEN), lambda i: (i, 0)),
                  pl.BlockSpec((_EB, _NOPEN), lambda i: (i, 0)),
                  pl.BlockSpec((_EB, 16), lambda i: (i, 0))]
                 + wspec
                 + [pl.BlockSpec((96, 96), lambda i: (0, 0)),
                    pl.BlockSpec((96, 96), lambda i: (0, 0))],
        out_specs=[pl.BlockSpec((_EB, _NOPEN), lambda i: (i, 0)),
                   pl.BlockSpec((_EB, _NOPEN), lambda i: (i, 0))],
        out_shape=[jax.ShapeDtypeStruct((_NEDGES, _NOPEN), _F32),
                   jax.ShapeDtypeStruct((_NEDGES, _NOPEN), _F32)],
    )(xi, xj, xep, *wargs, ke1, ke2)


def _concat_body(xn0_ref, dp_ref, dm_ref, a1_ref, a2_ref, out_ref):
    div = (dp_ref[0] + dp_ref[1]) - (dm_ref[0] + dm_ref[1])
    ave = jnp.maximum(a1_ref[0] + a1_ref[1], a2_ref[0] + a2_ref[1])
    out_ref[...] = jnp.concatenate([xn0_ref[...], div, ave], axis=1)


def _concat48(xn0, dp, dm, a1, a2):
    return pl.pallas_call(
        _concat_body,
        out_shape=jax.ShapeDtypeStruct((_NNODES, 3 * 16), _F32),
    )(xn0, dp, dm, a1, a2)


def _update_body(xn_ref, dp_ref, dm_ref, a1_ref, a2_ref, out_ref):
    div = (dp_ref[0] + dp_ref[1]) - (dm_ref[0] + dm_ref[1])
    ave = jnp.maximum(a1_ref[0] + a1_ref[1], a2_ref[0] + a2_ref[1])
    out_ref[...] = xn_ref[...] - _H * (div + ave)


def _update48(xn48, dp, dm, a1, a2):
    return pl.pallas_call(
        _update_body,
        out_shape=jax.ShapeDtypeStruct((_NNODES, _NOPEN), _F32),
    )(xn48, dp, dm, a1, a2)


def _close_body(kn_ref, xn_ref, out_ref):
    out_ref[...] = jnp.dot(kn_ref[...], xn_ref[...].T,
                           preferred_element_type=_F32, precision=lax.Precision.HIGHEST)


def _close(kn, xn48):
    return pl.pallas_call(
        _close_body,
        out_shape=jax.ShapeDtypeStruct((3, _NNODES), _F32),
    )(kn, xn48)


def _gather_rows(table, idx):
    return table[idx]


def _scatter_partials(vals_d, vals_a, i_idx, j_idx, ch):
    zero = jnp.zeros((_NNODES, ch), _F32)
    dp = zero.at[i_idx].add(vals_d)
    dm = zero.at[j_idx].add(vals_d)
    a1 = zero.at[i_idx].add(vals_a)
    a2 = zero.at[j_idx].add(vals_a)
    pad = jnp.zeros((1, _NNODES, ch), _F32)
    return (jnp.concatenate([dp[None], pad], 0),
            jnp.concatenate([dm[None], pad], 0),
            jnp.concatenate([a1[None], pad], 0),
            jnp.concatenate([a2[None], pad], 0))


def kernel(xn, xe, edge_index, K1Nopen, K2Nopen, K1Eopen, K2Eopen, KE1, KE2,
           KNclose, fA_W1, fA_b1, fA_W2, fA_b2, fB_W1, fB_b1, fB_W2, fB_b2):
    i_idx = edge_index[0]
    j_idx = edge_index[1]

    xn0 = _node_embed(xn[0], K1Nopen, K2Nopen)
    xe_out, xep, wg1, wg2 = _edge_embed(
        xe[0], K1Eopen, K2Eopen, fA_W1, fA_b1, fA_W2, fA_b2)

    dp, dm, a1, a2 = _scatter_partials(wg1, wg2, i_idx, j_idx, 16)
    xn48 = _concat48(xn0, dp, dm, a1, a2)

    nlayer = KE1.shape[0]
    for l in range(nlayer):
        xi = _gather_rows(xn48, i_idx)
        xj = _gather_rows(xn48, j_idx)
        fb = [(fB_W1[4 * l + k], fB_b1[4 * l + k],
               fB_W2[4 * l + k], fB_b2[4 * l + k]) for k in range(4)]
        wgd, wga = _edge_layer(xi, xj, xep, fb, KE1[l], KE2[l])
        dp, dm, a1, a2 = _scatter_partials(wgd, wga, i_idx, j_idx, _NOPEN)
        xn48 = _update48(xn48, dp, dm, a1, a2)

    xn_close = _close(KNclose, xn48)
    return (xn_close[None], xe_out[None])


# speedup vs baseline: 1.1210x; 1.1210x over previous
import functools

import jax
import jax.numpy as jnp
from jax import lax
from jax.experimental import pallas as pl
from jax.experimental.pallas import tpu as pltpu
from jax.experimental.pallas import tpu_sc as plsc

_NNODES = 10000
_NEDGES = 320000
_NOPEN = 48
_H = 0.1

_EB = 2560
_F32 = jnp.float32


def _tvn(x, axis):
    x = x - jnp.mean(x, axis=axis, keepdims=True)
    return x / jnp.sqrt(jnp.sum(x * x, axis=axis, keepdims=True) + 1e-3)


def _node_body(xn_ref, k1_ref, k2_ref, out_ref):
    x = jnp.tanh(xn_ref[...])
    x = jnp.dot(k1_ref[...], x, preferred_element_type=_F32, precision=lax.Precision.HIGHEST)
    x = _tvn(x, 0)
    x = jnp.tanh(x)
    x = jnp.dot(k2_ref[...], x, preferred_element_type=_F32, precision=lax.Precision.HIGHEST)
    x = jnp.tanh(x)
    out_ref[...] = x.T


def _node_embed(xn, k1, k2):
    return pl.pallas_call(
        _node_body,
        out_shape=jax.ShapeDtypeStruct((_NNODES, 16), _F32),
    )(xn, k1, k2)


def _edge_body(xe_ref, k1_ref, k2_ref,
               w1a_ref, b1a_ref, w2a_ref, b2a_ref,
               w1b_ref, b1b_ref, w2b_ref, b2b_ref,
               xeout_ref, xep_ref, wg1_ref, wg2_ref):
    x = jnp.tanh(xe_ref[...])
    x = jnp.dot(k1_ref[...], x, preferred_element_type=_F32, precision=lax.Precision.HIGHEST)
    x = _tvn(x, 0)
    x = jnp.tanh(x)
    x = jnp.dot(k2_ref[...], x, preferred_element_type=_F32, precision=lax.Precision.HIGHEST)
    x = jnp.tanh(x)
    xeout_ref[...] = x
    xt = x.T
    xep_ref[...] = xt

    def mlp(w1, b1, w2, b2):
        h = jnp.tanh(jnp.dot(xt, w1[...].T, preferred_element_type=_F32, precision=lax.Precision.HIGHEST)
                     + b1[...])
        return jnp.dot(h, w2[...].T, preferred_element_type=_F32, precision=lax.Precision.HIGHEST) + b2[...]

    wg1_ref[...] = mlp(w1a_ref, b1a_ref, w2a_ref, b2a_ref) * xt
    wg2_ref[...] = mlp(w1b_ref, b1b_ref, w2b_ref, b2b_ref) * xt


def _edge_embed(xe, k1, k2, fa_w1, fa_b1, fa_w2, fa_b2):
    nb = _NEDGES // _EB
    wspec = [pl.BlockSpec(s, lambda i: (0, 0))
             for s in [(16, 16), (16, 16),
                       (25, 16), (1, 25), (16, 25), (1, 16),
                       (25, 16), (1, 25), (16, 25), (1, 16)]]
    return pl.pallas_call(
        _edge_body,
        grid=(nb,),
        in_specs=[pl.BlockSpec((16, _EB), lambda i: (0, i))] + wspec,
        out_specs=[pl.BlockSpec((16, _EB), lambda i: (0, i)),
                   pl.BlockSpec((_EB, 16), lambda i: (i, 0)),
                   pl.BlockSpec((_EB, 16), lambda i: (i, 0)),
                   pl.BlockSpec((_EB, 16), lambda i: (i, 0))],
        out_shape=[jax.ShapeDtypeStruct((16, _NEDGES), _F32),
                   jax.ShapeDtypeStruct((_NEDGES, 16), _F32),
                   jax.ShapeDtypeStruct((_NEDGES, 16), _F32),
                   jax.ShapeDtypeStruct((_NEDGES, 16), _F32)],
    )(xe, k1, k2,
      fa_w1[0], fa_b1[0][None], fa_w2[0], fa_b2[0][None],
      fa_w1[1], fa_b1[1][None], fa_w2[1], fa_b2[1][None])


def _edgea_body(xi_ref, xj_ref, xep_ref,
                g_w1, g_b1, g_w2, g_b2,
                a_w1, a_b1, a_w2, a_b2,
                d_w1, d_b1, d_w2, d_b2,
                v_w1, v_b1, v_w2, v_b2,
                ke1_ref, ke2_ref,
                wgd_ref, wga_ref):
    xt = xep_ref[...]

    def mlp(w1, b1, w2, b2):
        h = jnp.tanh(jnp.dot(xt, w1[...].T, preferred_element_type=_F32, precision=lax.Precision.HIGHEST)
                     + b1[...])
        return jnp.dot(h, w2[...].T, preferred_element_type=_F32, precision=lax.Precision.HIGHEST) + b2[...]

    grad_x = mlp(g_w1, g_b1, g_w2, g_b2) * xi_ref[...]
    int_x = mlp(a_w1, a_b1, a_w2, a_b2) * xj_ref[...]
    e = jnp.concatenate([grad_x, int_x], axis=1)
    e = jnp.tanh(e)
    e = jnp.dot(e, ke1_ref[...].T, preferred_element_type=_F32, precision=lax.Precision.HIGHEST)
    e = _tvn(e, 1)
    e = jnp.tanh(e)
    e = jnp.dot(e, ke2_ref[...].T, preferred_element_type=_F32, precision=lax.Precision.HIGHEST)
    e = jnp.tanh(e)
    wgd_ref[...] = mlp(d_w1, d_b1, d_w2, d_b2) * e[:, :_NOPEN]
    wga_ref[...] = mlp(v_w1, v_b1, v_w2, v_b2) * e[:, _NOPEN:]


def _edge_layer(xi, xj, xep, fb, ke1, ke2):
    nb = _NEDGES // _EB
    wspec = []
    wargs = []
    for (w1, b1, w2, b2) in fb:
        wspec += [pl.BlockSpec((25, 16), lambda i: (0, 0)),
                  pl.BlockSpec((1, 25), lambda i: (0, 0)),
                  pl.BlockSpec((_NOPEN, 25), lambda i: (0, 0)),
                  pl.BlockSpec((1, _NOPEN), lambda i: (0, 0))]
        wargs += [w1, b1[None], w2, b2[None]]
    return pl.pallas_call(
        _edgea_body,
        grid=(nb,),
        in_specs=[pl.BlockSpec((_EB, _NOPEN), lambda i: (i, 0)),
                  pl.BlockSpec((_EB, _NOPEN), lambda i: (i, 0)),
                  pl.BlockSpec((_EB, 16), lambda i: (i, 0))]
                 + wspec
                 + [pl.BlockSpec((96, 96), lambda i: (0, 0)),
                    pl.BlockSpec((96, 96), lambda i: (0, 0))],
        out_specs=[pl.BlockSpec((_EB, _NOPEN), lambda i: (i, 0)),
                   pl.BlockSpec((_EB, _NOPEN), lambda i: (i, 0))],
        out_shape=[jax.ShapeDtypeStruct((_NEDGES, _NOPEN), _F32),
                   jax.ShapeDtypeStruct((_NEDGES, _NOPEN), _F32)],
    )(xi, xj, xep, *wargs, ke1, ke2)


def _concat_body(xn0_ref, dp_ref, dm_ref, a1_ref, a2_ref, out_ref):
    div = ((dp_ref[0, :_NNODES] + dp_ref[1, :_NNODES])
           - (dm_ref[0, :_NNODES] + dm_ref[1, :_NNODES]))
    ave = jnp.maximum(a1_ref[0, :_NNODES] + a1_ref[1, :_NNODES],
                      a2_ref[0, :_NNODES] + a2_ref[1, :_NNODES])
    pad = jnp.zeros((_NNODES, 128 - 3 * 16), _F32)
    out_ref[...] = jnp.concatenate([xn0_ref[...], div, ave, pad], axis=1)


def _concat48(xn0, dp, dm, a1, a2):
    return pl.pallas_call(
        _concat_body,
        out_shape=jax.ShapeDtypeStruct((_NNODES, 128), _F32),
    )(xn0, dp, dm, a1, a2)


def _update_body(xn_ref, dp_ref, dm_ref, a1_ref, a2_ref, out_ref):
    div = ((dp_ref[0, :_NNODES] + dp_ref[1, :_NNODES])
           - (dm_ref[0, :_NNODES] + dm_ref[1, :_NNODES]))
    ave = jnp.maximum(a1_ref[0, :_NNODES] + a1_ref[1, :_NNODES],
                      a2_ref[0, :_NNODES] + a2_ref[1, :_NNODES])
    upd = xn_ref[:, :_NOPEN] - _H * (div + ave)
    pad = jnp.zeros((_NNODES, 128 - _NOPEN), _F32)
    out_ref[...] = jnp.concatenate([upd, pad], axis=1)


def _update48(xn48, dp, dm, a1, a2):
    return pl.pallas_call(
        _update_body,
        out_shape=jax.ShapeDtypeStruct((_NNODES, 128), _F32),
    )(xn48, dp, dm, a1, a2)


def _close_body(kn_ref, xn_ref, out_ref):
    out_ref[...] = jnp.dot(kn_ref[...], xn_ref[:, :_NOPEN].T,
                           preferred_element_type=_F32, precision=lax.Precision.HIGHEST)


def _close(kn, xn48):
    return pl.pallas_call(
        _close_body,
        out_shape=jax.ShapeDtypeStruct((3, _NNODES), _F32),
    )(kn, xn48)


_NW = 32
_EPW = _NEDGES // _NW
_G = 80
_NCH = _EPW // _G
_NPAD = 10240
_RPS = _NPAD // 16


def _sc_mesh():
    return plsc.VectorSubcoreMesh(core_axis_name="c", subcore_axis_name="s")


def _sc_gather(tab, ii3, jj3):
    @functools.partial(
        pl.kernel, mesh=_sc_mesh(),
        out_type=(jax.ShapeDtypeStruct((_NEDGES, _NOPEN), _F32),
                  jax.ShapeDtypeStruct((_NEDGES, _NOPEN), _F32)),
        scratch_types=[pltpu.VMEM((_NCH, _G), jnp.int32),
                       pltpu.VMEM((_NCH, _G), jnp.int32),
                       pltpu.VMEM((_G, 128), _F32),
                       pltpu.VMEM((_G, 128), _F32),
                       pltpu.VMEM((_G, _NOPEN), _F32),
                       pltpu.VMEM((_G, _NOPEN), _F32),
                       pltpu.SemaphoreType.DMA,
                       pltpu.SemaphoreType.DMA],
    )
    def k(tab_hbm, ii_hbm, jj_hbm, d_hbm, a_hbm,
          ii_v, jj_v, xi_v, xj_v, d_v, a_v, si, sj):
        wid = lax.axis_index("s") * 2 + lax.axis_index("c")
        pltpu.sync_copy(ii_hbm.at[wid], ii_v)
        pltpu.sync_copy(jj_hbm.at[wid], jj_v)
        base = wid * _EPW

        def body(c, carry):
            cpi = pltpu.async_copy(tab_hbm.at[ii_v.at[c]], xi_v, si)
            cpj = pltpu.async_copy(tab_hbm.at[jj_v.at[c]], xj_v, sj)
            cpi.wait()
            cpj.wait()

            def rbody(r, carry2):
                for kk in range(_NOPEN // 16):
                    slk = pl.ds(kk * 16, 16)
                    a = xi_v[r, slk]
                    b = xj_v[r, slk]
                    d_v[r, slk] = a - b
                    a_v[r, slk] = (a + b) * 0.5
                return carry2

            lax.fori_loop(0, _G, rbody, 0)
            out_sl = pl.ds(base + c * _G, _G)
            pltpu.sync_copy(d_v, d_hbm.at[out_sl])
            pltpu.sync_copy(a_v, a_hbm.at[out_sl])
            return carry

        lax.fori_loop(0, _NCH, body, 0)

    return k(tab, ii3, jj3)


def _scatter_partials(vals, ii1, jj1, ch):
    zero = jnp.zeros((_NPAD, ch), _F32)
    pi = zero.at[ii1].add(vals)
    pj = zero.at[jj1].add(vals)
    pad = jnp.zeros((1, _NPAD, ch), _F32)
    return (jnp.concatenate([pi[None], pad], 0),
            jnp.concatenate([pj[None], pad], 0))


def kernel(xn, xe, edge_index, K1Nopen, K2Nopen, K1Eopen, K2Eopen, KE1, KE2,
           KNclose, fA_W1, fA_b1, fA_W2, fA_b2, fB_W1, fB_b1, fB_W2, fB_b2):
    ii1 = edge_index[0]
    jj1 = edge_index[1]
    ii3 = ii1.reshape(_NW, _NCH, _G)
    jj3 = jj1.reshape(_NW, _NCH, _G)

    xn0 = _node_embed(xn[0], K1Nopen, K2Nopen)
    xe_out, xep, wg1, wg2 = _edge_embed(
        xe[0], K1Eopen, K2Eopen, fA_W1, fA_b1, fA_W2, fA_b2)

    dp, dm = _scatter_partials(wg1, ii1, jj1, 16)
    a1, a2 = _scatter_partials(wg2, ii1, jj1, 16)
    xn48 = _concat48(xn0, dp, dm, a1, a2)

    nlayer = KE1.shape[0]
    for l in range(nlayer):
        xi, xj = _sc_gather(xn48, ii3, jj3)
        fb = [(fB_W1[4 * l + k], fB_b1[4 * l + k],
               fB_W2[4 * l + k], fB_b2[4 * l + k]) for k in range(4)]
        wgd, wga = _edge_layer(xi, xj, xep, fb, KE1[l], KE2[l])
        dp, dm = _scatter_partials(wgd, ii1, jj1, _NOPEN)
        a1, a2 = _scatter_partials(wga, ii1, jj1, _NOPEN)
        xn48 = _update48(xn48, dp, dm, a1, a2)

    xn_close = _close(KNclose, xn48)
    return (xn_close[None], xe_out[None])


# speedup vs baseline: 1.1379x; 1.0151x over previous
import functools

import jax
import jax.numpy as jnp
from jax import lax
from jax.experimental import pallas as pl
from jax.experimental.pallas import tpu as pltpu
from jax.experimental.pallas import tpu_sc as plsc

_NNODES = 10000
_NEDGES = 320000
_NOPEN = 48
_H = 0.1

_EB = 2560
_F32 = jnp.float32


def _tvn(x, axis):
    x = x - jnp.mean(x, axis=axis, keepdims=True)
    return x / jnp.sqrt(jnp.sum(x * x, axis=axis, keepdims=True) + 1e-3)


def _node_body(xn_ref, k1_ref, k2_ref, out_ref):
    x = jnp.tanh(xn_ref[...])
    x = jnp.dot(k1_ref[...], x, preferred_element_type=_F32, precision=lax.Precision.HIGHEST)
    x = _tvn(x, 0)
    x = jnp.tanh(x)
    x = jnp.dot(k2_ref[...], x, preferred_element_type=_F32, precision=lax.Precision.HIGHEST)
    x = jnp.tanh(x)
    out_ref[...] = x.T


def _node_embed(xn, k1, k2):
    return pl.pallas_call(
        _node_body,
        out_shape=jax.ShapeDtypeStruct((_NNODES, 16), _F32),
    )(xn, k1, k2)


def _edge_body(xe_ref, k1_ref, k2_ref,
               w1a_ref, b1a_ref, w2a_ref, b2a_ref,
               w1b_ref, b1b_ref, w2b_ref, b2b_ref,
               xeout_ref, xep_ref, wg1_ref, wg2_ref):
    x = jnp.tanh(xe_ref[...])
    x = jnp.dot(k1_ref[...], x, preferred_element_type=_F32, precision=lax.Precision.HIGHEST)
    x = _tvn(x, 0)
    x = jnp.tanh(x)
    x = jnp.dot(k2_ref[...], x, preferred_element_type=_F32, precision=lax.Precision.HIGHEST)
    x = jnp.tanh(x)
    xeout_ref[...] = x
    xt = x.T
    xep_ref[...] = xt

    def mlp(w1, b1, w2, b2):
        h = jnp.tanh(jnp.dot(xt, w1[...].T, preferred_element_type=_F32, precision=lax.Precision.HIGHEST)
                     + b1[...])
        return jnp.dot(h, w2[...].T, preferred_element_type=_F32, precision=lax.Precision.HIGHEST) + b2[...]

    wg1_ref[...] = mlp(w1a_ref, b1a_ref, w2a_ref, b2a_ref) * xt
    wg2_ref[...] = mlp(w1b_ref, b1b_ref, w2b_ref, b2b_ref) * xt


def _edge_embed(xe, k1, k2, fa_w1, fa_b1, fa_w2, fa_b2):
    nb = _NEDGES // _EB
    wspec = [pl.BlockSpec(s, lambda i: (0, 0))
             for s in [(16, 16), (16, 16),
                       (25, 16), (1, 25), (16, 25), (1, 16),
                       (25, 16), (1, 25), (16, 25), (1, 16)]]
    return pl.pallas_call(
        _edge_body,
        grid=(nb,),
        in_specs=[pl.BlockSpec((16, _EB), lambda i: (0, i))] + wspec,
        out_specs=[pl.BlockSpec((16, _EB), lambda i: (0, i)),
                   pl.BlockSpec((_EB, 16), lambda i: (i, 0)),
                   pl.BlockSpec((_EB, 16), lambda i: (i, 0)),
                   pl.BlockSpec((_EB, 16), lambda i: (i, 0))],
        out_shape=[jax.ShapeDtypeStruct((16, _NEDGES), _F32),
                   jax.ShapeDtypeStruct((_NEDGES, 16), _F32),
                   jax.ShapeDtypeStruct((_NEDGES, 16), _F32),
                   jax.ShapeDtypeStruct((_NEDGES, 16), _F32)],
    )(xe, k1, k2,
      fa_w1[0], fa_b1[0][None], fa_w2[0], fa_b2[0][None],
      fa_w1[1], fa_b1[1][None], fa_w2[1], fa_b2[1][None])


def _edgea_body(xi_ref, xj_ref, xep_ref,
                g_w1, g_b1, g_w2, g_b2,
                a_w1, a_b1, a_w2, a_b2,
                d_w1, d_b1, d_w2, d_b2,
                v_w1, v_b1, v_w2, v_b2,
                ke1_ref, ke2_ref,
                wgd_ref, wga_ref):
    xt = xep_ref[...]

    def mlp(w1, b1, w2, b2):
        h = jnp.tanh(jnp.dot(xt, w1[...].T, preferred_element_type=_F32, precision=lax.Precision.HIGHEST)
                     + b1[...])
        return jnp.dot(h, w2[...].T, preferred_element_type=_F32, precision=lax.Precision.HIGHEST) + b2[...]

    grad_x = mlp(g_w1, g_b1, g_w2, g_b2) * xi_ref[...]
    int_x = mlp(a_w1, a_b1, a_w2, a_b2) * xj_ref[...]
    e = jnp.concatenate([grad_x, int_x], axis=1)
    e = jnp.tanh(e)
    e = jnp.dot(e, ke1_ref[...].T, preferred_element_type=_F32, precision=lax.Precision.HIGHEST)
    e = _tvn(e, 1)
    e = jnp.tanh(e)
    e = jnp.dot(e, ke2_ref[...].T, preferred_element_type=_F32, precision=lax.Precision.HIGHEST)
    e = jnp.tanh(e)
    wgd_ref[...] = mlp(d_w1, d_b1, d_w2, d_b2) * e[:, :_NOPEN]
    wga_ref[...] = mlp(v_w1, v_b1, v_w2, v_b2) * e[:, _NOPEN:]


def _edge_layer(xi, xj, xep, fb, ke1, ke2):
    nb = _NEDGES // _EB
    wspec = []
    wargs = []
    for (w1, b1, w2, b2) in fb:
        wspec += [pl.BlockSpec((25, 16), lambda i: (0, 0)),
                  pl.BlockSpec((1, 25), lambda i: (0, 0)),
                  pl.BlockSpec((_NOPEN, 25), lambda i: (0, 0)),
                  pl.BlockSpec((1, _NOPEN), lambda i: (0, 0))]
        wargs += [w1, b1[None], w2, b2[None]]
    return pl.pallas_call(
        _edgea_body,
        grid=(nb,),
        in_specs=[pl.BlockSpec((_EB, _NOPEN), lambda i: (i, 0)),
                  pl.BlockSpec((_EB, _NOPEN), lambda i: (i, 0)),
                  pl.BlockSpec((_EB, 16), lambda i: (i, 0))]
                 + wspec
                 + [pl.BlockSpec((96, 96), lambda i: (0, 0)),
                    pl.BlockSpec((96, 96), lambda i: (0, 0))],
        out_specs=[pl.BlockSpec((_EB, _NOPEN), lambda i: (i, 0)),
                   pl.BlockSpec((_EB, _NOPEN), lambda i: (i, 0))],
        out_shape=[jax.ShapeDtypeStruct((_NEDGES, _NOPEN), _F32),
                   jax.ShapeDtypeStruct((_NEDGES, _NOPEN), _F32)],
    )(xi, xj, xep, *wargs, ke1, ke2)


def _concat_body(xn0_ref, dp_ref, dm_ref, a1_ref, a2_ref, out_ref):
    div = ((dp_ref[0, :_NNODES] + dp_ref[1, :_NNODES])
           - (dm_ref[0, :_NNODES] + dm_ref[1, :_NNODES]))
    ave = jnp.maximum(a1_ref[0, :_NNODES] + a1_ref[1, :_NNODES],
                      a2_ref[0, :_NNODES] + a2_ref[1, :_NNODES])
    pad = jnp.zeros((_NNODES, 128 - 3 * 16), _F32)
    out_ref[...] = jnp.concatenate([xn0_ref[...], div, ave, pad], axis=1)


def _concat48(xn0, dp, dm, a1, a2):
    return pl.pallas_call(
        _concat_body,
        out_shape=jax.ShapeDtypeStruct((_NNODES, 128), _F32),
    )(xn0, dp, dm, a1, a2)


def _update_body(xn_ref, dp_ref, dm_ref, a1_ref, a2_ref, out_ref):
    div = ((dp_ref[0, :_NNODES] + dp_ref[1, :_NNODES])
           - (dm_ref[0, :_NNODES] + dm_ref[1, :_NNODES]))
    ave = jnp.maximum(a1_ref[0, :_NNODES] + a1_ref[1, :_NNODES],
                      a2_ref[0, :_NNODES] + a2_ref[1, :_NNODES])
    upd = xn_ref[:, :_NOPEN] - _H * (div + ave)
    pad = jnp.zeros((_NNODES, 128 - _NOPEN), _F32)
    out_ref[...] = jnp.concatenate([upd, pad], axis=1)


def _update48(xn48, dp, dm, a1, a2):
    return pl.pallas_call(
        _update_body,
        out_shape=jax.ShapeDtypeStruct((_NNODES, 128), _F32),
    )(xn48, dp, dm, a1, a2)


def _close_body(kn_ref, xn_ref, out_ref):
    out_ref[...] = jnp.dot(kn_ref[...], xn_ref[:, :_NOPEN].T,
                           preferred_element_type=_F32, precision=lax.Precision.HIGHEST)


def _close(kn, xn48):
    return pl.pallas_call(
        _close_body,
        out_shape=jax.ShapeDtypeStruct((3, _NNODES), _F32),
    )(kn, xn48)


_NW = 32
_EPW = _NEDGES // _NW
_G = 80
_NCH = _EPW // _G
_NPAD = 10240
_RPS = _NPAD // 16


def _sc_mesh():
    return plsc.VectorSubcoreMesh(core_axis_name="c", subcore_axis_name="s")


def _sc_gather(tab, ii3, jj3):
    @functools.partial(
        pl.kernel, mesh=_sc_mesh(),
        out_type=(jax.ShapeDtypeStruct((_NEDGES, _NOPEN), _F32),
                  jax.ShapeDtypeStruct((_NEDGES, _NOPEN), _F32)),
        scratch_types=[pltpu.VMEM((_NCH, _G), jnp.int32),
                       pltpu.VMEM((_NCH, _G), jnp.int32),
                       pltpu.VMEM((_G, 128), _F32),
                       pltpu.VMEM((_G, 128), _F32),
                       pltpu.VMEM((_G, 128), _F32),
                       pltpu.VMEM((_G, 128), _F32),
                       pltpu.VMEM((_G, _NOPEN), _F32),
                       pltpu.VMEM((_G, _NOPEN), _F32),
                       pltpu.SemaphoreType.DMA,
                       pltpu.SemaphoreType.DMA,
                       pltpu.SemaphoreType.DMA,
                       pltpu.SemaphoreType.DMA],
    )
    def k(tab_hbm, ii_hbm, jj_hbm, d_hbm, a_hbm,
          ii_v, jj_v, xi0_v, xj0_v, xi1_v, xj1_v, d_v, a_v,
          si0, sj0, si1, sj1):
        wid = lax.axis_index("s") * 2 + lax.axis_index("c")
        pltpu.sync_copy(ii_hbm.at[wid], ii_v)
        pltpu.sync_copy(jj_hbm.at[wid], jj_v)
        base = wid * _EPW

        def fire(c, xi_v, xj_v, si, sj):
            pltpu.async_copy(tab_hbm.at[ii_v.at[c]], xi_v, si)
            pltpu.async_copy(tab_hbm.at[jj_v.at[c]], xj_v, sj)

        def drain_process(c, xi_v, xj_v, si, sj):
            pltpu.make_async_copy(tab_hbm.at[ii_v.at[c]], xi_v, si).wait()
            pltpu.make_async_copy(tab_hbm.at[jj_v.at[c]], xj_v, sj).wait()

            def rbody(r, carry2):
                for kk in range(_NOPEN // 16):
                    slk = pl.ds(kk * 16, 16)
                    a = xi_v[r, slk]
                    b = xj_v[r, slk]
                    d_v[r, slk] = a - b
                    a_v[r, slk] = (a + b) * 0.5
                return carry2

            lax.fori_loop(0, _G, rbody, 0)
            out_sl = pl.ds(base + c * _G, _G)
            pltpu.sync_copy(d_v, d_hbm.at[out_sl])
            pltpu.sync_copy(a_v, a_hbm.at[out_sl])

        fire(0, xi0_v, xj0_v, si0, sj0)

        def body(t, carry):
            c0 = 2 * t
            fire(c0 + 1, xi1_v, xj1_v, si1, sj1)
            drain_process(c0, xi0_v, xj0_v, si0, sj0)
            fire(c0 + 2, xi0_v, xj0_v, si0, sj0)
            drain_process(c0 + 1, xi1_v, xj1_v, si1, sj1)
            return carry

        lax.fori_loop(0, (_NCH - 1) // 2, body, 0)
        drain_process(_NCH - 1, xi0_v, xj0_v, si0, sj0)

    return k(tab, ii3, jj3)


def _scatter_partials(vals, ii1, jj1, ch):
    zero = jnp.zeros((_NPAD, ch), _F32)
    pi = zero.at[ii1].add(vals)
    pj = zero.at[jj1].add(vals)
    pad = jnp.zeros((1, _NPAD, ch), _F32)
    return (jnp.concatenate([pi[None], pad], 0),
            jnp.concatenate([pj[None], pad], 0))


def kernel(xn, xe, edge_index, K1Nopen, K2Nopen, K1Eopen, K2Eopen, KE1, KE2,
           KNclose, fA_W1, fA_b1, fA_W2, fA_b2, fB_W1, fB_b1, fB_W2, fB_b2):
    ii1 = edge_index[0]
    jj1 = edge_index[1]
    ii3 = ii1.reshape(_NW, _NCH, _G)
    jj3 = jj1.reshape(_NW, _NCH, _G)

    xn0 = _node_embed(xn[0], K1Nopen, K2Nopen)
    xe_out, xep, wg1, wg2 = _edge_embed(
        xe[0], K1Eopen, K2Eopen, fA_W1, fA_b1, fA_W2, fA_b2)

    dp, dm = _scatter_partials(wg1, ii1, jj1, 16)
    a1, a2 = _scatter_partials(wg2, ii1, jj1, 16)
    xn48 = _concat48(xn0, dp, dm, a1, a2)

    nlayer = KE1.shape[0]
    for l in range(nlayer):
        xi, xj = _sc_gather(xn48, ii3, jj3)
        fb = [(fB_W1[4 * l + k], fB_b1[4 * l + k],
               fB_W2[4 * l + k], fB_b2[4 * l + k]) for k in range(4)]
        wgd, wga = _edge_layer(xi, xj, xep, fb, KE1[l], KE2[l])
        dp, dm = _scatter_partials(wgd, ii1, jj1, _NOPEN)
        a1, a2 = _scatter_partials(wga, ii1, jj1, _NOPEN)
        xn48 = _update48(xn48, dp, dm, a1, a2)

    xn_close = _close(KNclose, xn48)
    return (xn_close[None], xe_out[None])
